# baseline (speedup 1.0000x reference)
import jax
import jax.numpy as jnp
from jax import lax
from jax.experimental import pallas as pl
from jax.experimental.pallas import tpu as pltpu
from jax.experimental.pallas import tpu_sc as plsc

_BATCH = 16384
_K = 16
_NROWS = 1000000

_info = plsc.get_sparse_core_info()
_NC, _NS, _L = _info.num_cores, _info.num_subcores, _info.num_lanes
_NW = _NC * _NS
_BPW = _BATCH // _NW
_NBLK = _BPW // _L


def _ncf_body(u_hbm, v_hbm, wt_hbm, ht_hbm, lin_hbm, out_hbm,
              uidx_v, vidx_v, ubuf_v, vbuf_v, lin_v, out_v, sem):
    wid = lax.axis_index("s") * _NC + lax.axis_index("c")
    base = wid * _BPW

    pltpu.sync_copy(lin_hbm, lin_v)
    pltpu.sync_copy(u_hbm.at[pl.ds(base, _BPW)], uidx_v)
    pltpu.sync_copy(v_hbm.at[pl.ds(base, _BPW)], vidx_v)

    copies = []
    for k in range(_K):
        copies.append(pltpu.async_copy(
            wt_hbm.at[pl.ds(k * _NROWS, _NROWS)].at[uidx_v], ubuf_v.at[k], sem))
        copies.append(pltpu.async_copy(
            ht_hbm.at[pl.ds(k * _NROWS, _NROWS)].at[vidx_v], vbuf_v.at[k], sem))
    for c in copies:
        c.wait()

    wu_vec = lin_v[pl.ds(0, _L)]
    wv_vec = lin_v[pl.ds(_K, _L)]
    wb_vec = lin_v[pl.ds(2 * _K, _L)]
    wk = [wu_vec[k] for k in range(_K)] + [wv_vec[k] for k in range(_K)]
    lb = wb_vec[0]

    def blk_body(blk, carry):
        rbase = blk * _L
        acc = jnp.full((_L,), 0.0, jnp.float32)
        for k in range(_K):
            acc = acc + ubuf_v[k, pl.ds(rbase, _L)] * wk[k]
            acc = acc + vbuf_v[k, pl.ds(rbase, _L)] * wk[_K + k]
        z = acc + lb
        out_v[pl.ds(rbase, _L)] = 1.0 / (1.0 + jnp.exp(-z))
        return carry

    lax.fori_loop(0, _NBLK, blk_body, 0)

    pltpu.sync_copy(out_v, out_hbm.at[pl.ds(base, _BPW)])


_ncf_sc = pl.kernel(
    _ncf_body,
    mesh=plsc.VectorSubcoreMesh(core_axis_name="c", subcore_axis_name="s"),
    out_type=jax.ShapeDtypeStruct((_BATCH,), jnp.float32),
    scratch_types=[
        pltpu.VMEM((_BPW,), jnp.int32),
        pltpu.VMEM((_BPW,), jnp.int32),
        pltpu.VMEM((_K, _BPW), jnp.float32),
        pltpu.VMEM((_K, _BPW), jnp.float32),
        pltpu.VMEM((48,), jnp.float32),
        pltpu.VMEM((_BPW,), jnp.float32),
        pltpu.SemaphoreType.DMA,
    ],
    compiler_params=pltpu.CompilerParams(
        needs_layout_passes=False, use_tc_tiling_on_sc=False),
)


@jax.jit
def kernel(x, W, H, lin_w, lin_b):
    u_idx = x[:, 0]
    v_idx = x[:, 1]
    wt = W.T.reshape(-1)
    ht = H.T.reshape(-1)
    lin_all = jnp.concatenate(
        [lin_w.reshape(-1), lin_b.reshape(-1), jnp.zeros((15,), jnp.float32)])
    return _ncf_sc(u_idx, v_idx, wt, ht, lin_all)


# speedup vs baseline: 19.1948x; 19.1948x over previous
import jax
import jax.numpy as jnp
from jax import lax
from jax.experimental import pallas as pl
from jax.experimental.pallas import tpu as pltpu
from jax.experimental.pallas import tpu_sc as plsc

_BATCH = 16384
_K = 16
_NROWS = 1000000

_info = plsc.get_sparse_core_info()
_NC, _NS, _L = _info.num_cores, _info.num_subcores, _info.num_lanes
_NW = _NC * _NS
_BPW = _BATCH // _NW
_NBLK = _BPW // _L


def _ncf_body(u_hbm, v_hbm, wt_hbm, ht_hbm, lin_hbm, out_hbm,
              uidx_v, vidx_v, uwin_v, vwin_v, lin_v, out_v, sem):
    wid = lax.axis_index("s") * _NC + lax.axis_index("c")
    base = wid * _BPW

    pltpu.sync_copy(lin_hbm, lin_v)
    pltpu.sync_copy(u_hbm.at[pl.ds(base, _BPW)], uidx_v)
    pltpu.sync_copy(v_hbm.at[pl.ds(base, _BPW)], vidx_v)

    wu_vec = lin_v[pl.ds(0, _L)]
    wv_vec = lin_v[pl.ds(_K, _L)]
    wb_vec = lin_v[pl.ds(2 * _K, _L)]
    wk = [wu_vec[k] for k in range(_K)] + [wv_vec[k] for k in range(_K)]
    lb = wb_vec[0]

    lane = lax.iota(jnp.int32, _L)

    def blk_body(blk, carry):
        rbase = blk * _L
        uc = uidx_v[pl.ds(rbase, _L)]
        vc = vidx_v[pl.ds(rbase, _L)]
        copies = []
        for j in range(_L):
            ualign = pl.multiple_of((uc[j] >> 7) * 128, 128)
            valign = pl.multiple_of((vc[j] >> 7) * 128, 128)
            copies.append(pltpu.async_copy(
                wt_hbm.at[:, pl.ds(ualign, 128)], uwin_v.at[j], sem))
            copies.append(pltpu.async_copy(
                ht_hbm.at[:, pl.ds(valign, 128)], vwin_v.at[j], sem))
        for c in copies:
            c.wait()

        ucol = uc & 127
        vcol = vc & 127
        acc = jnp.full((_L,), 0.0, jnp.float32)
        for k in range(_K):
            plane = jnp.full((_L,), k, jnp.int32)
            uval = plsc.load_gather(uwin_v, [lane, plane, ucol])
            vval = plsc.load_gather(vwin_v, [lane, plane, vcol])
            acc = acc + uval * wk[k] + vval * wk[_K + k]
        z = acc + lb
        out_v[pl.ds(rbase, _L)] = 1.0 / (1.0 + jnp.exp(-z))
        return carry

    lax.fori_loop(0, _NBLK, blk_body, 0)

    pltpu.sync_copy(out_v, out_hbm.at[pl.ds(base, _BPW)])


_ncf_sc = pl.kernel(
    _ncf_body,
    mesh=plsc.VectorSubcoreMesh(core_axis_name="c", subcore_axis_name="s"),
    out_type=jax.ShapeDtypeStruct((_BATCH,), jnp.float32),
    scratch_types=[
        pltpu.VMEM((_BPW,), jnp.int32),
        pltpu.VMEM((_BPW,), jnp.int32),
        pltpu.VMEM((_L, _K, 128), jnp.float32),
        pltpu.VMEM((_L, _K, 128), jnp.float32),
        pltpu.VMEM((48,), jnp.float32),
        pltpu.VMEM((_BPW,), jnp.float32),
        pltpu.SemaphoreType.DMA,
    ],
    compiler_params=pltpu.CompilerParams(needs_layout_passes=False),
)


@jax.jit
def kernel(x, W, H, lin_w, lin_b):
    u_idx = x[:, 0]
    v_idx = x[:, 1]
    wt = W.T
    ht = H.T
    lin_all = jnp.concatenate(
        [lin_w.reshape(-1), lin_b.reshape(-1), jnp.zeros((15,), jnp.float32)])
    return _ncf_sc(u_idx, v_idx, wt, ht, lin_all)


# speedup vs baseline: 19.4227x; 1.0119x over previous
import jax
import jax.numpy as jnp
from jax import lax
from jax.experimental import pallas as pl
from jax.experimental.pallas import tpu as pltpu
from jax.experimental.pallas import tpu_sc as plsc

_BATCH = 16384
_K = 16
_NROWS = 1000000

_info = plsc.get_sparse_core_info()
_NC, _NS, _L = _info.num_cores, _info.num_subcores, _info.num_lanes
_NW = _NC * _NS
_BPW = _BATCH // _NW
_NBLK = _BPW // _L


def _ncf_body(u_hbm, v_hbm, wt_hbm, ht_hbm, lin_hbm, out_hbm,
              uidx_v, vidx_v, uwin_v, vwin_v, lin_v, out_v, sem_u, sem_v):
    wid = lax.axis_index("s") * _NC + lax.axis_index("c")
    base = wid * _BPW

    pltpu.sync_copy(lin_hbm, lin_v)
    pltpu.sync_copy(u_hbm.at[pl.ds(base, _BPW)], uidx_v)
    pltpu.sync_copy(v_hbm.at[pl.ds(base, _BPW)], vidx_v)

    wu_vec = lin_v[pl.ds(0, _L)]
    wv_vec = lin_v[pl.ds(_K, _L)]
    wb_vec = lin_v[pl.ds(2 * _K, _L)]
    wk = [wu_vec[k] for k in range(_K)] + [wv_vec[k] for k in range(_K)]
    lb = wb_vec[0]

    lane = lax.iota(jnp.int32, _L)

    def fire(idx_ref, tbl, win, sem, blk):
        c = idx_ref[pl.ds(blk * _L, _L)]
        for j in range(_L):
            a = pl.multiple_of((c[j] >> 7) * 128, 128)
            pltpu.async_copy(tbl.at[:, pl.ds(a, 128)], win.at[j], sem)

    def drain(tbl, win, sem):
        for j in range(_L):
            pltpu.make_async_copy(tbl.at[:, pl.ds(0, 128)], win.at[j], sem).wait()

    fire(uidx_v, wt_hbm, uwin_v, sem_u, 0)

    def blk_body(blk, carry):
        rbase = blk * _L
        fire(vidx_v, ht_hbm, vwin_v, sem_v, blk)

        drain(wt_hbm, uwin_v, sem_u)
        ucol = uidx_v[pl.ds(rbase, _L)] & 127
        acc = jnp.full((_L,), 0.0, jnp.float32)
        for k in range(_K):
            plane = jnp.full((_L,), k, jnp.int32)
            acc = acc + plsc.load_gather(uwin_v, [lane, plane, ucol]) * wk[k]

        @pl.when(blk < _NBLK - 1)
        def _():
            fire(uidx_v, wt_hbm, uwin_v, sem_u, blk + 1)

        drain(ht_hbm, vwin_v, sem_v)
        vcol = vidx_v[pl.ds(rbase, _L)] & 127
        for k in range(_K):
            plane = jnp.full((_L,), k, jnp.int32)
            acc = acc + plsc.load_gather(vwin_v, [lane, plane, vcol]) * wk[_K + k]

        z = acc + lb
        out_v[pl.ds(rbase, _L)] = 1.0 / (1.0 + jnp.exp(-z))
        return carry

    lax.fori_loop(0, _NBLK, blk_body, 0)

    pltpu.sync_copy(out_v, out_hbm.at[pl.ds(base, _BPW)])


_ncf_sc = pl.kernel(
    _ncf_body,
    mesh=plsc.VectorSubcoreMesh(core_axis_name="c", subcore_axis_name="s"),
    out_type=jax.ShapeDtypeStruct((_BATCH,), jnp.float32),
    scratch_types=[
        pltpu.VMEM((_BPW,), jnp.int32),
        pltpu.VMEM((_BPW,), jnp.int32),
        pltpu.VMEM((_L, _K, 128), jnp.float32),
        pltpu.VMEM((_L, _K, 128), jnp.float32),
        pltpu.VMEM((48,), jnp.float32),
        pltpu.VMEM((_BPW,), jnp.float32),
        pltpu.SemaphoreType.DMA,
        pltpu.SemaphoreType.DMA,
    ],
    compiler_params=pltpu.CompilerParams(needs_layout_passes=False),
)


@jax.jit
def kernel(x, W, H, lin_w, lin_b):
    u_idx = x[:, 0]
    v_idx = x[:, 1]
    wt = W.T
    ht = H.T
    lin_all = jnp.concatenate(
        [lin_w.reshape(-1), lin_b.reshape(-1), jnp.zeros((15,), jnp.float32)])
    return _ncf_sc(u_idx, v_idx, wt, ht, lin_all)


# speedup vs baseline: 19.5601x; 1.0071x over previous
import jax
import jax.numpy as jnp
from jax import lax
from jax.experimental import pallas as pl
from jax.experimental.pallas import tpu as pltpu
from jax.experimental.pallas import tpu_sc as plsc

_BATCH = 16384
_K = 16
_NROWS = 1000000

_info = plsc.get_sparse_core_info()
_NC, _NS, _L = _info.num_cores, _info.num_subcores, _info.num_lanes
_NW = _NC * _NS
_BPW = _BATCH // _NW
_NBLK = _BPW // _L


def _ncf_body(xt_hbm, wt_hbm, ht_hbm, lin_hbm, out_hbm,
              uidx_v, vidx_v, uwin_v, vwin_v, lin_v, out_v, sem_u, sem_v):
    wid = lax.axis_index("s") * _NC + lax.axis_index("c")
    base = wid * _BPW

    pltpu.sync_copy(lin_hbm, lin_v)
    pltpu.sync_copy(xt_hbm.at[0, pl.ds(base, _BPW)], uidx_v)
    pltpu.sync_copy(xt_hbm.at[1, pl.ds(base, _BPW)], vidx_v)

    wu_vec = lin_v[pl.ds(0, _L)]
    wv_vec = lin_v[pl.ds(_K, _L)]
    wb_vec = lin_v[pl.ds(2 * _K, _L)]
    wk = [wu_vec[k] for k in range(_K)] + [wv_vec[k] for k in range(_K)]
    lb = wb_vec[0]

    lane = lax.iota(jnp.int32, _L)

    def fire(idx_ref, tbl, win, sem, blk):
        c = idx_ref[pl.ds(blk * _L, _L)]
        for j in range(_L):
            a = pl.multiple_of((c[j] >> 7) * 128, 128)
            pltpu.async_copy(tbl.at[:, pl.ds(a, 128)], win.at[j], sem)

    def drain(tbl, win, sem):
        for j in range(_L):
            pltpu.make_async_copy(tbl.at[:, pl.ds(0, 128)], win.at[j], sem).wait()

    fire(uidx_v, wt_hbm, uwin_v, sem_u, 0)

    def blk_body(blk, carry):
        rbase = blk * _L
        fire(vidx_v, ht_hbm, vwin_v, sem_v, blk)

        drain(wt_hbm, uwin_v, sem_u)
        ucol = uidx_v[pl.ds(rbase, _L)] & 127
        acc = jnp.full((_L,), 0.0, jnp.float32)
        for k in range(_K):
            plane = jnp.full((_L,), k, jnp.int32)
            acc = acc + plsc.load_gather(uwin_v, [lane, plane, ucol]) * wk[k]

        @pl.when(blk < _NBLK - 1)
        def _():
            fire(uidx_v, wt_hbm, uwin_v, sem_u, blk + 1)

        drain(ht_hbm, vwin_v, sem_v)
        vcol = vidx_v[pl.ds(rbase, _L)] & 127
        for k in range(_K):
            plane = jnp.full((_L,), k, jnp.int32)
            acc = acc + plsc.load_gather(vwin_v, [lane, plane, vcol]) * wk[_K + k]

        z = acc + lb
        out_v[pl.ds(rbase, _L)] = 1.0 / (1.0 + jnp.exp(-z))
        return carry

    lax.fori_loop(0, _NBLK, blk_body, 0)

    pltpu.sync_copy(out_v, out_hbm.at[pl.ds(base, _BPW)])


_ncf_sc = pl.kernel(
    _ncf_body,
    mesh=plsc.VectorSubcoreMesh(core_axis_name="c", subcore_axis_name="s"),
    out_type=jax.ShapeDtypeStruct((_BATCH,), jnp.float32),
    scratch_types=[
        pltpu.VMEM((_BPW,), jnp.int32),
        pltpu.VMEM((_BPW,), jnp.int32),
        pltpu.VMEM((_L, _K, 128), jnp.float32),
        pltpu.VMEM((_L, _K, 128), jnp.float32),
        pltpu.VMEM((48,), jnp.float32),
        pltpu.VMEM((_BPW,), jnp.float32),
        pltpu.SemaphoreType.DMA,
        pltpu.SemaphoreType.DMA,
    ],
    compiler_params=pltpu.CompilerParams(needs_layout_passes=False),
)


@jax.jit
def kernel(x, W, H, lin_w, lin_b):
    xt = x.T
    wt = W.T
    ht = H.T
    lin_all = jnp.concatenate(
        [lin_w.reshape(-1), lin_b.reshape(-1), jnp.zeros((15,), jnp.float32)])
    return _ncf_sc(xt, wt, ht, lin_all)
